# baseline (speedup 1.0000x reference)
import jax
import jax.numpy as jnp
from jax.experimental import pallas as pl

_B, _D, _H, _W, _K = 16, 64, 24, 24, 1024
_HW = _H * _W


def _tree_argmin(v):
    k = v.shape[0]
    rel = None
    while k > 1:
        h = k // 2
        vlo, vhi = v[:h], v[h:]
        take = vhi < vlo
        v = jnp.where(take, vhi, vlo)
        if rel is None:
            rel = jnp.where(take, jnp.int32(h), jnp.int32(0))
        else:
            rel = jnp.where(take, rel[h:] + jnp.int32(h), rel[:h])
        k = h
    return rel


_G = 8


def _vq_kernel(x_ref, w_ref, out_ref, idx_ref):
    w = w_ref[...]
    w_sq = jnp.sum(w * w, axis=0, keepdims=True).reshape(_K, 1)
    w_sq_m = jnp.broadcast_to(w_sq, (_K, 4 * _HW))
    iota_k = jax.lax.broadcasted_iota(jnp.int32, (_K, 4 * _HW), 0)
    for g in range(0, _G, 4):
        x = jnp.concatenate(
            [x_ref[g], x_ref[g + 1], x_ref[g + 2], x_ref[g + 3]], axis=1)
        scores = jax.lax.dot_general(
            w, x, (((0,), (0,)), ((), ())),
            preferred_element_type=jnp.float32)
        d2 = w_sq_m - 2.0 * scores
        idx = _tree_argmin(d2)
        onehot = (iota_k == idx).astype(jnp.float32)
        res = jax.lax.dot_general(
            w, onehot, (((1,), (0,)), ((), ())),
            preferred_element_type=jnp.float32)
        for j in range(4):
            out_ref[g + j] = res[:, j * _HW:(j + 1) * _HW]
            idx_ref[g + j, 0] = idx[0, j * _HW:(j + 1) * _HW]


def kernel(x, weight):
    x3 = x.reshape(_B, _D, _HW)
    result, idx = pl.pallas_call(
        _vq_kernel,
        grid=(_B // _G,),
        in_specs=[
            pl.BlockSpec((_G, _D, _HW), lambda b: (b, 0, 0)),
            pl.BlockSpec((_D, _K), lambda b: (0, 0)),
        ],
        out_specs=[
            pl.BlockSpec((_G, _D, _HW), lambda b: (b, 0, 0)),
            pl.BlockSpec((_G, 1, _HW), lambda b: (b, 0, 0)),
        ],
        out_shape=[
            jax.ShapeDtypeStruct((_B, _D, _HW), jnp.float32),
            jax.ShapeDtypeStruct((_B, 1, _HW), jnp.int32),
        ],
    )(x3, weight)
    return result.reshape(_B, _D, _H, _W), idx.reshape(_B, _H, _W)


# speedup vs baseline: 1.0020x; 1.0020x over previous
import jax
import jax.numpy as jnp
from jax.experimental import pallas as pl

_B, _D, _H, _W, _K = 16, 64, 24, 24, 1024
_HW = _H * _W


def _tree_argmin(v):
    k = v.shape[0]
    rel = None
    while k > 1:
        h = k // 2
        vlo, vhi = v[:h], v[h:]
        take = vhi < vlo
        v = jnp.where(take, vhi, vlo)
        if rel is None:
            rel = jnp.where(take, jnp.int32(h), jnp.int32(0))
        else:
            rel = jnp.where(take, rel[h:] + jnp.int32(h), rel[:h])
        k = h
    return rel


_G = 8


def _vq_kernel(x_ref, w_ref, out_ref, idx_ref):
    w = w_ref[...]
    w_sq = jnp.sum(w * w, axis=0, keepdims=True).reshape(_K, 1)
    w_sq_m = jnp.broadcast_to(w_sq, (_K, 2 * _HW))
    iota_k = jax.lax.broadcasted_iota(jnp.int32, (_K, 2 * _HW), 0)
    for g in range(0, _G, 2):
        x = jnp.concatenate([x_ref[g], x_ref[g + 1]], axis=1)
        scores = jax.lax.dot_general(
            w, x, (((0,), (0,)), ((), ())),
            preferred_element_type=jnp.float32)
        d2 = w_sq_m - 2.0 * scores
        idx = _tree_argmin(d2)
        onehot = (iota_k == idx).astype(jnp.float32)
        res = jax.lax.dot_general(
            w, onehot, (((1,), (0,)), ((), ())),
            preferred_element_type=jnp.float32)
        out_ref[g] = res[:, :_HW]
        out_ref[g + 1] = res[:, _HW:]
        idx_ref[g, 0] = idx[0, :_HW]
        idx_ref[g + 1, 0] = idx[0, _HW:]


def kernel(x, weight):
    x3 = x.reshape(_B, _D, _HW)
    result, idx = pl.pallas_call(
        _vq_kernel,
        grid=(_B // _G,),
        in_specs=[
            pl.BlockSpec((_G, _D, _HW), lambda b: (b, 0, 0)),
            pl.BlockSpec((_D, _K), lambda b: (0, 0)),
        ],
        out_specs=[
            pl.BlockSpec((_G, _D, _HW), lambda b: (b, 0, 0)),
            pl.BlockSpec((_G, 1, _HW), lambda b: (b, 0, 0)),
        ],
        out_shape=[
            jax.ShapeDtypeStruct((_B, _D, _HW), jnp.float32),
            jax.ShapeDtypeStruct((_B, 1, _HW), jnp.int32),
        ],
    )(x3, weight)
    return result.reshape(_B, _D, _H, _W), idx.reshape(_B, _H, _W)
